# Initial kernel scaffold; baseline (speedup 1.0000x reference)
import jax
import jax.numpy as jnp
from jax.experimental import pallas as pl


def kernel(x, edge_index, W1_0, b1_0, W1_1, b1_1, fc_W, fc_b, bn_g, bn_b, fc2_W, fc2_b):
    raise NotImplementedError("write your pallas kernel here")


# speedup vs baseline: 6.1324x; 6.1324x over previous
import functools
import math

import jax
import jax.numpy as jnp
from jax import lax
from jax.experimental import pallas as pl
from jax.experimental.pallas import tpu as pltpu
from jax.experimental.pallas import tpu_sc as plsc

N = 10000
NP = 10240
E = 320000
D = 128
ALPHA = 0.1
EPS = 1e-5
BETA1 = float(math.log(2.0))
BETA2 = float(math.log(1.5))

NC = 2
NS = 16
NW = NC * NS
EPW = E // NW
CH = 125
NCH = EPW // CH
ROWS_PER_TILE = NP // NS

_mesh = plsc.VectorSubcoreMesh(core_axis_name="c", subcore_axis_name="s")


@functools.partial(
    pl.kernel,
    out_type=jax.ShapeDtypeStruct((NW, NP), jnp.float32),
    mesh=_mesh,
    scratch_types=[
        pltpu.VMEM((NP,), jnp.float32),
        pltpu.VMEM((EPW,), jnp.int32),
    ],
    compiler_params=pltpu.CompilerParams(needs_layout_passes=False),
)
def _sc_degree(dst_hbm, degp_hbm, deg_v, idx_v):
    cid = lax.axis_index("c")
    sid = lax.axis_index("s")
    wid = sid * NC + cid

    z16 = jnp.zeros((16,), jnp.float32)

    def zero_body(i, c):
        deg_v[pl.ds(i * 16, 16)] = z16
        return c

    lax.fori_loop(0, NP // 16, zero_body, 0)

    pltpu.sync_copy(dst_hbm.at[pl.ds(wid * EPW, EPW)], idx_v)

    ones16 = jnp.ones((16,), jnp.float32)

    def acc_body(k, c):
        idx16 = idx_v[pl.ds(k * 16, 16)]
        plsc.addupdate_scatter(deg_v, [idx16], ones16)
        return c

    lax.fori_loop(0, EPW // 16, acc_body, 0)

    pltpu.sync_copy(deg_v, degp_hbm.at[wid])


@functools.partial(
    pl.kernel,
    out_type=jax.ShapeDtypeStruct((NC, NP, D), jnp.float32),
    mesh=_mesh,
    scratch_types=[
        pltpu.VMEM((NCH, CH), jnp.int32),
        pltpu.VMEM((NCH, CH), jnp.int32),
        pltpu.VMEM((CH, D), jnp.float32),
        pltpu.VMEM((80, D), jnp.float32),
        pltpu.VMEM_SHARED((NP, D), jnp.float32),
        pltpu.SemaphoreType.DMA,
    ],
    compiler_params=pltpu.CompilerParams(needs_layout_passes=False),
)
def _sc_spmm(feat_hbm, src_hbm, dst_hbm, aggp_hbm, sidx, didx, rows, zb, agg_sh, sem):
    cid = lax.axis_index("c")
    sid = lax.axis_index("s")
    wid = sid * NC + cid

    z16 = jnp.zeros((16,), jnp.float32)

    def zb_row(r, c):
        def zb_col(q, c2):
            zb[r, pl.ds(q * 16, 16)] = z16
            return c2
        return lax.fori_loop(0, D // 16, zb_col, c)

    lax.fori_loop(0, 80, zb_row, 0)

    base = sid * ROWS_PER_TILE

    def zcopy(t, c):
        pltpu.sync_copy(zb, agg_sh.at[pl.ds(base + t * 80, 80)])
        return c

    lax.fori_loop(0, ROWS_PER_TILE // 80, zcopy, 0)
    plsc.subcore_barrier()

    pltpu.sync_copy(src_hbm.at[pl.ds(wid * NCH, NCH)], sidx)
    pltpu.sync_copy(dst_hbm.at[pl.ds(wid * NCH, NCH)], didx)

    def chunk_body(j, c):
        pltpu.async_copy(feat_hbm.at[sidx.at[j]], rows, sem).wait()
        pltpu.sync_copy(rows, agg_sh.at[didx.at[j]], add=True)
        return c

    lax.fori_loop(0, NCH, chunk_body, 0)
    plsc.subcore_barrier()

    pltpu.sync_copy(
        agg_sh.at[pl.ds(base, ROWS_PER_TILE)],
        aggp_hbm.at[cid].at[pl.ds(base, ROWS_PER_TILE)],
    )


RB = 1280
GRID = NP // RB


def _tc_norm_feat_body(degp_ref, x_ref, norm_ref, feat_ref):
    d = jnp.sum(degp_ref[...], axis=0)
    nm = lax.rsqrt(jnp.maximum(d, 1.0))
    norm_ref[...] = nm
    feat_ref[...] = x_ref[...] * nm


def _tc_layer_body(beta, aggp_ref, norm_ref, x_ref, w_ref, b_ref, feat_ref):
    nm = norm_ref[...]
    agg = (aggp_ref[0] + aggp_ref[1]) * nm
    rst = (1.0 - ALPHA) * agg + ALPHA * x_ref[...]
    h = (1.0 - beta) * rst + beta * jnp.dot(
        rst, w_ref[...], preferred_element_type=jnp.float32) + b_ref[...]
    feat_ref[...] = h * nm


def _tc_layer2_head_body(aggp_ref, norm_ref, x_ref, w_ref, b_ref, fcw_ref,
                         fcb_ref, t_ref, s_ref, sq_ref):
    i = pl.program_id(0)
    nm = norm_ref[...]
    agg = (aggp_ref[0] + aggp_ref[1]) * nm
    rst = (1.0 - ALPHA) * agg + ALPHA * x_ref[...]
    h = (1.0 - BETA2) * rst + BETA2 * jnp.dot(
        rst, w_ref[...], preferred_element_type=jnp.float32) + b_ref[...]
    t = jnp.maximum(
        jnp.dot(h, fcw_ref[...], preferred_element_type=jnp.float32)
        + fcb_ref[...], 0.0)
    t_ref[...] = t
    rowid = i * RB + lax.broadcasted_iota(jnp.int32, (RB, 1), 0)
    tm = jnp.where(rowid < N, t, 0.0)
    s = jnp.sum(tm, axis=0, keepdims=True)
    sq = jnp.sum(tm * tm, axis=0, keepdims=True)

    @pl.when(i == 0)
    def _():
        s_ref[...] = s
        sq_ref[...] = sq

    @pl.when(i > 0)
    def _():
        s_ref[...] += s
        sq_ref[...] += sq


def _tc_bn_out_body(t_ref, s_ref, sq_ref, g_ref, bb_ref, w2_ref, b2_ref,
                    out_ref):
    mu = s_ref[...] * (1.0 / N)
    var = sq_ref[...] * (1.0 / N) - mu * mu
    inv = lax.rsqrt(var + EPS)
    y = (t_ref[...] - mu) * inv * g_ref[...] + bb_ref[...]
    out_ref[...] = jnp.maximum(
        jnp.dot(y, w2_ref[...], preferred_element_type=jnp.float32)
        + b2_ref[...], 0.0)


def _full(shape):
    return pl.BlockSpec(shape, lambda i: tuple(0 for _ in shape))


def kernel(x, edge_index, W1_0, b1_0, W1_1, b1_1, fc_W, fc_b, bn_g, bn_b,
           fc2_W, fc2_b):
    src = edge_index[0].reshape(NW * NCH, CH)
    dst_flat = edge_index[1]
    dst = dst_flat.reshape(NW * NCH, CH)
    xp = jnp.pad(x, ((0, NP - N), (0, 0)))

    degp = _sc_degree(dst_flat)
    degp3 = degp.reshape(NW, NP, 1)

    norm, feat0 = pl.pallas_call(
        _tc_norm_feat_body,
        grid=(GRID,),
        in_specs=[
            pl.BlockSpec((NW, RB, 1), lambda i: (0, i, 0)),
            pl.BlockSpec((RB, D), lambda i: (i, 0)),
        ],
        out_specs=[
            pl.BlockSpec((RB, 1), lambda i: (i, 0)),
            pl.BlockSpec((RB, D), lambda i: (i, 0)),
        ],
        out_shape=[
            jax.ShapeDtypeStruct((NP, 1), jnp.float32),
            jax.ShapeDtypeStruct((NP, D), jnp.float32),
        ],
    )(degp3, xp)

    aggp1 = _sc_spmm(feat0, src, dst)

    feat1 = pl.pallas_call(
        functools.partial(_tc_layer_body, BETA1),
        grid=(GRID,),
        in_specs=[
            pl.BlockSpec((NC, RB, D), lambda i: (0, i, 0)),
            pl.BlockSpec((RB, 1), lambda i: (i, 0)),
            pl.BlockSpec((RB, D), lambda i: (i, 0)),
            _full((D, D)),
            _full((1, D)),
        ],
        out_specs=pl.BlockSpec((RB, D), lambda i: (i, 0)),
        out_shape=jax.ShapeDtypeStruct((NP, D), jnp.float32),
    )(aggp1, norm, xp, W1_0, b1_0.reshape(1, D))

    aggp2 = _sc_spmm(feat1, src, dst)

    t, s, sq = pl.pallas_call(
        _tc_layer2_head_body,
        grid=(GRID,),
        in_specs=[
            pl.BlockSpec((NC, RB, D), lambda i: (0, i, 0)),
            pl.BlockSpec((RB, 1), lambda i: (i, 0)),
            pl.BlockSpec((RB, D), lambda i: (i, 0)),
            _full((D, D)),
            _full((1, D)),
            _full((D, 2 * D)),
            _full((1, 2 * D)),
        ],
        out_specs=[
            pl.BlockSpec((RB, 2 * D), lambda i: (i, 0)),
            _full((1, 2 * D)),
            _full((1, 2 * D)),
        ],
        out_shape=[
            jax.ShapeDtypeStruct((NP, 2 * D), jnp.float32),
            jax.ShapeDtypeStruct((1, 2 * D), jnp.float32),
            jax.ShapeDtypeStruct((1, 2 * D), jnp.float32),
        ],
    )(aggp2, norm, xp, W1_1, b1_1.reshape(1, D), fc_W, fc_b.reshape(1, 2 * D))

    out = pl.pallas_call(
        _tc_bn_out_body,
        grid=(GRID,),
        in_specs=[
            pl.BlockSpec((RB, 2 * D), lambda i: (i, 0)),
            _full((1, 2 * D)),
            _full((1, 2 * D)),
            _full((1, 2 * D)),
            _full((1, 2 * D)),
            _full((2 * D, D)),
            _full((1, D)),
        ],
        out_specs=pl.BlockSpec((RB, D), lambda i: (i, 0)),
        out_shape=jax.ShapeDtypeStruct((NP, D), jnp.float32),
    )(t, s, sq, bn_g.reshape(1, 2 * D), bn_b.reshape(1, 2 * D), fc2_W,
      fc2_b.reshape(1, D))

    return out[:N]
